# Initial kernel scaffold; baseline (speedup 1.0000x reference)
import jax
import jax.numpy as jnp
from jax.experimental import pallas as pl


def kernel(z, x, edge_index, edge_weight, edge_attr, emb, Wd, bd, Wc, bc):
    raise NotImplementedError("write your pallas kernel here")


# speedup vs baseline: 1.5382x; 1.5382x over previous
import functools
from math import pi as PI

import jax
import jax.numpy as jnp
from jax import lax
from jax.experimental import pallas as pl
from jax.experimental.pallas import tpu as pltpu
from jax.experimental.pallas import tpu_sc as plsc

H = 128
NRBF = 16
NZ = 100
STOP = 5.0

NC = 2
NS = 16
NW = NC * NS


def _zgather_body(z_hbm, row_hbm, zrow_hbm, idx_v, out_v, sem):
    n_chunks = row_hbm.shape[0]
    wid = lax.axis_index("s") * NC + lax.axis_index("c")
    base_chunks = n_chunks // NW
    extra = n_chunks - base_chunks * NW
    start = wid * base_chunks + jnp.minimum(wid, extra)
    count = base_chunks + (wid < extra).astype(jnp.int32)

    def chunk(j, _):
        cid = start + j
        pltpu.sync_copy(row_hbm.at[cid], idx_v)
        pltpu.async_copy(z_hbm.at[idx_v], out_v, sem).wait()
        pltpu.sync_copy(out_v, zrow_hbm.at[cid])
        return 0

    lax.fori_loop(0, count, chunk, 0)


def _sc_zgather(z, row2d):
    n_chunks, ck = row2d.shape
    mesh = plsc.VectorSubcoreMesh(core_axis_name="c", subcore_axis_name="s",
                                  num_cores=NC, num_subcores=NS)
    return pl.kernel(
        _zgather_body,
        out_type=jax.ShapeDtypeStruct((n_chunks, ck), jnp.int32),
        mesh=mesh,
        scratch_types=[
            pltpu.VMEM((ck,), jnp.int32),
            pltpu.VMEM((ck,), jnp.int32),
            pltpu.SemaphoreType.DMA,
        ],
    )(z, row2d)


def _msg_body(attr_ref, w_ref, row_ref, col_ref, zr_ref, embp_ref, wd_ref,
              bd_ref, out_ref):
    w = w_ref[...]
    C = 0.5 * (jnp.cos(w * (PI / STOP)) + 1.0)
    C = C * (w < STOP).astype(jnp.float32)
    C = C * (row_ref[...] != col_ref[...]).astype(jnp.float32)
    B = attr_ref.shape[0]
    ids = jax.lax.broadcasted_iota(jnp.int32, (B, H), 1)
    oh = (zr_ref[...] == ids).astype(jnp.float32) * C
    embrow = jnp.dot(oh, embp_ref[...], preferred_element_type=jnp.float32)
    attrw = jnp.dot(attr_ref[...], wd_ref[...],
                    preferred_element_type=jnp.float32) + bd_ref[...]
    out_ref[...] = embrow * attrw


def _tc_msg(attr, weight, row, col, zrow, embp, Wd, bd):
    E = attr.shape[0]
    BE = 2560
    nb = E // BE
    col1 = lambda i: (i, 0)
    return pl.pallas_call(
        _msg_body,
        grid=(nb,),
        in_specs=[
            pl.BlockSpec((BE, NRBF), col1),
            pl.BlockSpec((BE, 1), col1),
            pl.BlockSpec((BE, 1), col1),
            pl.BlockSpec((BE, 1), col1),
            pl.BlockSpec((BE, 1), col1),
            pl.BlockSpec((H, H), lambda i: (0, 0)),
            pl.BlockSpec((NRBF, H), lambda i: (0, 0)),
            pl.BlockSpec((1, H), lambda i: (0, 0)),
        ],
        out_specs=pl.BlockSpec((BE, H), col1),
        out_shape=jax.ShapeDtypeStruct((E, H), jnp.float32),
        compiler_params=pltpu.CompilerParams(
            dimension_semantics=("arbitrary",)),
    )(attr, weight.reshape(E, 1), row.reshape(E, 1), col.reshape(E, 1),
      zrow.reshape(E, 1), embp, Wd, bd.reshape(1, H))


def _scatter_body(col_hbm, msg_hbm, zer_hbm, out_hbm, agg_sh, col_v, upd_v,
                  sem):
    n_chunks = col_hbm.shape[0]
    N = zer_hbm.shape[0]
    rows_per_tile = N // NS
    cid_core = lax.axis_index("c")
    sid = lax.axis_index("s")
    wid = sid * NC + cid_core

    r0 = sid * rows_per_tile
    pltpu.sync_copy(zer_hbm.at[pl.ds(r0, rows_per_tile), :],
                    agg_sh.at[pl.ds(r0, rows_per_tile), :])
    plsc.subcore_barrier()

    base_chunks = n_chunks // NW
    extra = n_chunks - base_chunks * NW
    start = wid * base_chunks + jnp.minimum(wid, extra)
    count = base_chunks + (wid < extra).astype(jnp.int32)

    def chunk(j, _):
        cid = start + j
        pltpu.sync_copy(col_hbm.at[cid], col_v)
        pltpu.async_copy(msg_hbm.at[pl.ds(cid * 128, 128), :], upd_v,
                         sem).wait()
        pltpu.sync_copy(upd_v, agg_sh.at[col_v], add=True)
        return 0

    lax.fori_loop(0, count, chunk, 0)
    plsc.subcore_barrier()
    pltpu.sync_copy(agg_sh.at[pl.ds(r0, rows_per_tile), :],
                    out_hbm.at[cid_core, pl.ds(r0, rows_per_tile), :])


def _sc_scatter(col2d, msg, zer):
    N = zer.shape[0]
    mesh = plsc.VectorSubcoreMesh(core_axis_name="c", subcore_axis_name="s", num_cores=NC, num_subcores=NS)
    return pl.kernel(
        _scatter_body,
        out_type=jax.ShapeDtypeStruct((NC, N, H), jnp.float32),
        mesh=mesh,
        scratch_types=[
            pltpu.VMEM_SHARED((N, H), jnp.float32),
            pltpu.VMEM((128,), jnp.int32),
            pltpu.VMEM((128, H), jnp.float32),
            pltpu.SemaphoreType.DMA,
        ],
    )(col2d, msg, zer)


def _out_body(x_ref, p0_ref, p1_ref, wct_ref, wcb_ref, bc_ref, out_ref):
    agg = p0_ref[...] + p1_ref[...]
    out_ref[...] = (
        jnp.dot(x_ref[...], wct_ref[...], preferred_element_type=jnp.float32)
        + jnp.dot(agg, wcb_ref[...], preferred_element_type=jnp.float32)
        + bc_ref[...])


def _tc_out(x, p0, p1, WcT, WcB, bc):
    N = x.shape[0]
    BN = 2000
    nb = N // BN
    col1 = lambda i: (i, 0)
    return pl.pallas_call(
        _out_body,
        grid=(nb,),
        in_specs=[
            pl.BlockSpec((BN, H), col1),
            pl.BlockSpec((BN, H), col1),
            pl.BlockSpec((BN, H), col1),
            pl.BlockSpec((H, H), lambda i: (0, 0)),
            pl.BlockSpec((H, H), lambda i: (0, 0)),
            pl.BlockSpec((1, H), lambda i: (0, 0)),
        ],
        out_specs=pl.BlockSpec((BN, H), col1),
        out_shape=jax.ShapeDtypeStruct((N, H), jnp.float32),
        compiler_params=pltpu.CompilerParams(
            dimension_semantics=("arbitrary",)),
    )(x, p0, p1, WcT, WcB, bc.reshape(1, H))


def kernel(z, x, edge_index, edge_weight, edge_attr, emb, Wd, bd, Wc, bc):
    N = x.shape[0]
    E = edge_index.shape[1]
    row = edge_index[0]
    col = edge_index[1]

    zrow = _sc_zgather(z.astype(jnp.int32), row.reshape(E // 128, 128)).reshape(E)

    embp = jnp.zeros((H, H), jnp.float32).at[:NZ, :].set(emb)
    msg = _tc_msg(edge_attr, edge_weight, row, col, zrow, embp, Wd, bd)

    NPAD = 10240
    zer = jnp.zeros((NPAD, H), jnp.float32)
    partials = _sc_scatter(col.reshape(E // 128, 128), msg, zer)

    return _tc_out(x, partials[0], partials[1], Wc[:H], Wc[H:], bc)


# speedup vs baseline: 2.7032x; 1.7574x over previous
import functools
from math import pi as PI

import jax
import jax.numpy as jnp
from jax import lax
from jax.experimental import pallas as pl
from jax.experimental.pallas import tpu as pltpu
from jax.experimental.pallas import tpu_sc as plsc

H = 128
NRBF = 16
NZ = 100
STOP = 5.0

NC = 2
NS = 16
NW = NC * NS

CK = 80
CPW = 125


def _sc_mesh():
    return plsc.VectorSubcoreMesh(core_axis_name="c", subcore_axis_name="s",
                                  num_cores=NC, num_subcores=NS)


def _coef_body(w_ref, row_ref, col_ref, out_ref):
    w = w_ref[...]
    C = 0.5 * (jnp.cos(w * (PI / STOP)) + 1.0)
    C = C * (w < STOP).astype(jnp.float32)
    C = C * (row_ref[...] != col_ref[...]).astype(jnp.float32)
    out_ref[...] = C


def _tc_coef(weight, row, col):
    E = weight.shape[0]
    nr = E // H
    shp = (nr, H)
    full = pl.BlockSpec(shp, lambda: (0, 0))
    return pl.pallas_call(
        _coef_body,
        in_specs=[full, full, full],
        out_specs=full,
        out_shape=jax.ShapeDtypeStruct(shp, jnp.float32),
    )(weight.reshape(shp), row.reshape(shp), col.reshape(shp))


def _zgather_body(z_hbm, row_hbm, zrow_hbm, idx_v, out_v, sem):
    wid = lax.axis_index("s") * NC + lax.axis_index("c")
    pltpu.sync_copy(row_hbm.at[wid], idx_v)
    BURST = 25
    for b0 in range(0, CPW, BURST):
        descs = [
            pltpu.async_copy(z_hbm.at[idx_v.at[j]], out_v.at[j], sem)
            for j in range(b0, min(b0 + BURST, CPW))
        ]
        for d in descs:
            d.wait()
    pltpu.sync_copy(out_v, zrow_hbm.at[wid])


def _sc_zgather(z, row3):
    mesh = _sc_mesh()
    return pl.kernel(
        _zgather_body,
        out_type=jax.ShapeDtypeStruct((NW, CPW, CK), jnp.int32),
        mesh=mesh,
        scratch_types=[
            pltpu.VMEM((CPW, CK), jnp.int32),
            pltpu.VMEM((CPW, CK), jnp.int32),
            pltpu.SemaphoreType.DMA,
        ],
    )(z, row3)


def _msg_body(attr_ref, c_ref, zr_ref, embp_ref, wd_ref, bd_ref, out_ref):
    B = attr_ref.shape[0]
    ids = jax.lax.broadcasted_iota(jnp.int32, (B, H), 1)
    oh = (zr_ref[...] == ids).astype(jnp.float32) * c_ref[...]
    embrow = jnp.dot(oh, embp_ref[...], preferred_element_type=jnp.float32)
    attrw = jnp.dot(attr_ref[...], wd_ref[...],
                    preferred_element_type=jnp.float32) + bd_ref[...]
    out_ref[...] = embrow * attrw


def _tc_msg(attr, coef, zrow, embp, Wd, bd):
    E = attr.shape[0]
    BE = 2560
    nb = E // BE
    col1 = lambda i: (i, 0)
    return pl.pallas_call(
        _msg_body,
        grid=(nb,),
        in_specs=[
            pl.BlockSpec((BE, NRBF), col1),
            pl.BlockSpec((BE, 1), col1),
            pl.BlockSpec((BE, 1), col1),
            pl.BlockSpec((H, H), lambda i: (0, 0)),
            pl.BlockSpec((NRBF, H), lambda i: (0, 0)),
            pl.BlockSpec((1, H), lambda i: (0, 0)),
        ],
        out_specs=pl.BlockSpec((BE, H), col1),
        out_shape=jax.ShapeDtypeStruct((E, H), jnp.float32),
        compiler_params=pltpu.CompilerParams(
            dimension_semantics=("arbitrary",)),
    )(attr, coef.reshape(E, 1), zrow.reshape(E, 1), embp, Wd,
      bd.reshape(1, H))


def _scatter_body(col_hbm, msg_hbm, zer_hbm, out_hbm, agg_sh, col_v, upd0,
                  upd1, upd2, lsem0, lsem1, lsem2, ssem0, ssem1, ssem2):
    NP = zer_hbm.shape[0]
    rows_per_tile = NP // NS
    cid_core = lax.axis_index("c")
    sid = lax.axis_index("s")
    wid = sid * NC + cid_core

    r0 = sid * rows_per_tile
    pltpu.sync_copy(zer_hbm.at[pl.ds(r0, rows_per_tile), :],
                    agg_sh.at[pl.ds(r0, rows_per_tile), :])
    pltpu.sync_copy(col_hbm.at[wid], col_v)
    plsc.subcore_barrier()

    e0 = wid * (CPW * CK)
    bufs = (upd0, upd1, upd2)
    lsems = (lsem0, lsem1, lsem2)
    ssems = (ssem0, ssem1, ssem2)

    def load(j):
        return pltpu.async_copy(
            msg_hbm.at[pl.ds(e0 + j * CK, CK), :], bufs[j % 3], lsems[j % 3])

    def scat(j):
        return pltpu.async_copy(bufs[j % 3], agg_sh.at[col_v.at[j]],
                                ssems[j % 3], add=True)

    ld = [load(0), load(1), None]
    sc = [None, None, None]
    for j in range(CPW):
        b = j % 3
        ld[b].wait()
        sc[b] = scat(j)
        if j + 2 < CPW:
            b2 = (j + 2) % 3
            if sc[b2] is not None:
                sc[b2].wait()
            ld[b2] = load(j + 2)
    for d in sc:
        if d is not None:
            d.wait()

    plsc.subcore_barrier()
    pltpu.sync_copy(agg_sh.at[pl.ds(r0, rows_per_tile), :],
                    out_hbm.at[cid_core, pl.ds(r0, rows_per_tile), :])


def _sc_scatter(col3, msg, zer):
    NP = zer.shape[0]
    mesh = _sc_mesh()
    return pl.kernel(
        _scatter_body,
        out_type=jax.ShapeDtypeStruct((NC, NP, H), jnp.float32),
        mesh=mesh,
        scratch_types=[
            pltpu.VMEM_SHARED((NP, H), jnp.float32),
            pltpu.VMEM((CPW, CK), jnp.int32),
            pltpu.VMEM((CK, H), jnp.float32),
            pltpu.VMEM((CK, H), jnp.float32),
            pltpu.VMEM((CK, H), jnp.float32),
            pltpu.SemaphoreType.DMA,
            pltpu.SemaphoreType.DMA,
            pltpu.SemaphoreType.DMA,
            pltpu.SemaphoreType.DMA,
            pltpu.SemaphoreType.DMA,
            pltpu.SemaphoreType.DMA,
        ],
    )(col3, msg, zer)


def _out_body(x_ref, p0_ref, p1_ref, wct_ref, wcb_ref, bc_ref, out_ref):
    agg = p0_ref[...] + p1_ref[...]
    out_ref[...] = (
        jnp.dot(x_ref[...], wct_ref[...], preferred_element_type=jnp.float32)
        + jnp.dot(agg, wcb_ref[...], preferred_element_type=jnp.float32)
        + bc_ref[...])


def _tc_out(x, p0, p1, WcT, WcB, bc):
    N = x.shape[0]
    BN = 2000
    nb = N // BN
    col1 = lambda i: (i, 0)
    return pl.pallas_call(
        _out_body,
        grid=(nb,),
        in_specs=[
            pl.BlockSpec((BN, H), col1),
            pl.BlockSpec((BN, H), col1),
            pl.BlockSpec((BN, H), col1),
            pl.BlockSpec((H, H), lambda i: (0, 0)),
            pl.BlockSpec((H, H), lambda i: (0, 0)),
            pl.BlockSpec((1, H), lambda i: (0, 0)),
        ],
        out_specs=pl.BlockSpec((BN, H), col1),
        out_shape=jax.ShapeDtypeStruct((N, H), jnp.float32),
        compiler_params=pltpu.CompilerParams(
            dimension_semantics=("arbitrary",)),
    )(x, p0, p1, WcT, WcB, bc.reshape(1, H))


def kernel(z, x, edge_index, edge_weight, edge_attr, emb, Wd, bd, Wc, bc):
    N = x.shape[0]
    E = edge_index.shape[1]
    row = edge_index[0]
    col = edge_index[1]

    coef = _tc_coef(edge_weight, row, col)
    zrow = _sc_zgather(z.astype(jnp.int32), row.reshape(NW, CPW, CK))

    embp = jnp.zeros((H, H), jnp.float32).at[:NZ, :].set(emb)
    msg = _tc_msg(edge_attr, coef.reshape(E), zrow.reshape(E), embp, Wd, bd)

    NPAD = 10240
    zer = jnp.zeros((NPAD, H), jnp.float32)
    partials = _sc_scatter(col.reshape(NW, CPW, CK), msg, zer)

    return _tc_out(x, partials[0], partials[1], Wc[:H], Wc[H:], bc)


# speedup vs baseline: 6.3769x; 2.3590x over previous
import functools
from math import pi as PI

import jax
import jax.numpy as jnp
from jax import lax
from jax.experimental import pallas as pl
from jax.experimental.pallas import tpu as pltpu
from jax.experimental.pallas import tpu_sc as plsc

H = 128
NRBF = 16
NZ = 100
STOP = 5.0

NC = 2
NS = 16
NW = NC * NS

CK = 80
CPW = 125


def _sc_mesh():
    return plsc.VectorSubcoreMesh(core_axis_name="c", subcore_axis_name="s",
                                  num_cores=NC, num_subcores=NS)


def _coef_body(w_ref, row_ref, col_ref, out_ref):
    w = w_ref[...]
    C = 0.5 * (jnp.cos(w * (PI / STOP)) + 1.0)
    C = C * (w < STOP).astype(jnp.float32)
    C = C * (row_ref[...] != col_ref[...]).astype(jnp.float32)
    out_ref[...] = C


def _tc_coef(weight, row, col):
    E = weight.shape[0]
    nr = E // H
    shp = (nr, H)
    full = pl.BlockSpec(shp, lambda: (0, 0))
    return pl.pallas_call(
        _coef_body,
        in_specs=[full, full, full],
        out_specs=full,
        out_shape=jax.ShapeDtypeStruct(shp, jnp.float32),
    )(weight.reshape(shp), row.reshape(shp), col.reshape(shp))


def _zgather_body(z_hbm, row_hbm, zrow_hbm, idx_v, out_v, sem):
    wid = lax.axis_index("s") * NC + lax.axis_index("c")
    pltpu.sync_copy(row_hbm.at[wid], idx_v)
    BURST = 25
    for b0 in range(0, CPW, BURST):
        descs = [
            pltpu.async_copy(z_hbm.at[idx_v.at[j]], out_v.at[j], sem)
            for j in range(b0, min(b0 + BURST, CPW))
        ]
        for d in descs:
            d.wait()
    pltpu.sync_copy(out_v, zrow_hbm.at[wid])


def _sc_zgather(z, row3):
    mesh = _sc_mesh()
    return pl.kernel(
        _zgather_body,
        out_type=jax.ShapeDtypeStruct((NW, CPW, CK), jnp.int32),
        mesh=mesh,
        scratch_types=[
            pltpu.VMEM((CPW, CK), jnp.int32),
            pltpu.VMEM((CPW, CK), jnp.int32),
            pltpu.SemaphoreType.DMA,
        ],
    )(z, row3)


def _msg_body(zr_ref, c_ref, attrT_ref, embp_ref, wd_ref, bd_ref, out_ref):
    ng = zr_ref.shape[0]
    ids = jax.lax.broadcasted_iota(jnp.int32, (H, H), 0)
    dn = (((0,), (0,)), ((), ()))
    for g in range(ng):
        zrg = zr_ref[g:g + 1, :]
        cg = c_ref[g:g + 1, :]
        ohCT = (zrg == ids).astype(jnp.float32) * cg
        embrow = jax.lax.dot_general(ohCT, embp_ref[...], dn,
                                     preferred_element_type=jnp.float32)
        attrwT = attrT_ref[:, g * H:(g + 1) * H]
        attrw = jax.lax.dot_general(attrwT, wd_ref[...], dn,
                                    preferred_element_type=jnp.float32)
        out_ref[g * H:(g + 1) * H, :] = embrow * (attrw + bd_ref[...])


def _tc_msg(attrT2, zr2, c2, embp, Wd, bd):
    E2 = attrT2.shape[1]
    BE = 4096
    GR = BE // H
    nb = E2 // BE
    return pl.pallas_call(
        _msg_body,
        grid=(nb,),
        in_specs=[
            pl.BlockSpec((GR, H), lambda i: (i, 0)),
            pl.BlockSpec((GR, H), lambda i: (i, 0)),
            pl.BlockSpec((NRBF, BE), lambda i: (0, i)),
            pl.BlockSpec((H, H), lambda i: (0, 0)),
            pl.BlockSpec((NRBF, H), lambda i: (0, 0)),
            pl.BlockSpec((1, H), lambda i: (0, 0)),
        ],
        out_specs=pl.BlockSpec((BE, H), lambda i: (i, 0)),
        out_shape=jax.ShapeDtypeStruct((E2, H), jnp.float32),
        compiler_params=pltpu.CompilerParams(
            dimension_semantics=("arbitrary",)),
    )(zr2, c2, attrT2, embp, Wd, bd.reshape(1, H))


def _scatter_body(col_hbm, msg_hbm, zer_hbm, out_hbm, agg_sh, col_v, upd0,
                  upd1, upd2, lsem0, lsem1, lsem2, ssem0, ssem1, ssem2):
    NP = zer_hbm.shape[0]
    rows_per_tile = NP // NS
    cid_core = lax.axis_index("c")
    sid = lax.axis_index("s")
    wid = sid * NC + cid_core

    r0 = sid * rows_per_tile
    pltpu.sync_copy(zer_hbm.at[pl.ds(r0, rows_per_tile), :],
                    agg_sh.at[pl.ds(r0, rows_per_tile), :])
    pltpu.sync_copy(col_hbm.at[wid], col_v)
    plsc.subcore_barrier()

    e0 = wid * (CPW * CK)
    bufs = (upd0, upd1, upd2)
    lsems = (lsem0, lsem1, lsem2)
    ssems = (ssem0, ssem1, ssem2)

    def load(j):
        return pltpu.async_copy(
            msg_hbm.at[pl.ds(e0 + j * CK, CK), :], bufs[j % 3], lsems[j % 3])

    def scat(j):
        return pltpu.async_copy(bufs[j % 3], agg_sh.at[col_v.at[j]],
                                ssems[j % 3], add=True)

    ld = [load(0), load(1), None]
    sc = [None, None, None]
    for j in range(CPW):
        b = j % 3
        ld[b].wait()
        sc[b] = scat(j)
        if j + 2 < CPW:
            b2 = (j + 2) % 3
            if sc[b2] is not None:
                sc[b2].wait()
            ld[b2] = load(j + 2)
    for d in sc:
        if d is not None:
            d.wait()

    plsc.subcore_barrier()
    pltpu.sync_copy(agg_sh.at[pl.ds(r0, rows_per_tile), :],
                    out_hbm.at[cid_core, pl.ds(r0, rows_per_tile), :])


def _sc_scatter(col3, msg, zer):
    NP = zer.shape[0]
    mesh = _sc_mesh()
    return pl.kernel(
        _scatter_body,
        out_type=jax.ShapeDtypeStruct((NC, NP, H), jnp.float32),
        mesh=mesh,
        scratch_types=[
            pltpu.VMEM_SHARED((NP, H), jnp.float32),
            pltpu.VMEM((CPW, CK), jnp.int32),
            pltpu.VMEM((CK, H), jnp.float32),
            pltpu.VMEM((CK, H), jnp.float32),
            pltpu.VMEM((CK, H), jnp.float32),
            pltpu.SemaphoreType.DMA,
            pltpu.SemaphoreType.DMA,
            pltpu.SemaphoreType.DMA,
            pltpu.SemaphoreType.DMA,
            pltpu.SemaphoreType.DMA,
            pltpu.SemaphoreType.DMA,
        ],
    )(col3, msg, zer)


def _out_body(x_ref, p0_ref, p1_ref, wct_ref, wcb_ref, bc_ref, out_ref):
    agg = p0_ref[...] + p1_ref[...]
    out_ref[...] = (
        jnp.dot(x_ref[...], wct_ref[...], preferred_element_type=jnp.float32)
        + jnp.dot(agg, wcb_ref[...], preferred_element_type=jnp.float32)
        + bc_ref[...])


def _tc_out(x, p0, p1, WcT, WcB, bc):
    N = x.shape[0]
    BN = 2000
    nb = N // BN
    col1 = lambda i: (i, 0)
    return pl.pallas_call(
        _out_body,
        grid=(nb,),
        in_specs=[
            pl.BlockSpec((BN, H), col1),
            pl.BlockSpec((BN, H), col1),
            pl.BlockSpec((BN, H), col1),
            pl.BlockSpec((H, H), lambda i: (0, 0)),
            pl.BlockSpec((H, H), lambda i: (0, 0)),
            pl.BlockSpec((1, H), lambda i: (0, 0)),
        ],
        out_specs=pl.BlockSpec((BN, H), col1),
        out_shape=jax.ShapeDtypeStruct((N, H), jnp.float32),
        compiler_params=pltpu.CompilerParams(
            dimension_semantics=("arbitrary",)),
    )(x, p0, p1, WcT, WcB, bc.reshape(1, H))


def kernel(z, x, edge_index, edge_weight, edge_attr, emb, Wd, bd, Wc, bc):
    N = x.shape[0]
    E = edge_index.shape[1]
    row = edge_index[0]
    col = edge_index[1]

    coef = _tc_coef(edge_weight, row, col)
    zrow = _sc_zgather(z.astype(jnp.int32), row.reshape(NW, CPW, CK))

    E2 = 327680
    padE = E2 - E
    zr2 = jnp.concatenate(
        [zrow.reshape(E), jnp.zeros((padE,), jnp.int32)]).reshape(E2 // H, H)
    c2 = jnp.concatenate(
        [coef.reshape(E), jnp.zeros((padE,), jnp.float32)]).reshape(E2 // H, H)
    attrT2 = jnp.pad(edge_attr.T, ((0, 0), (0, padE)))

    embp = jnp.zeros((H, H), jnp.float32).at[:NZ, :].set(emb)
    msg = _tc_msg(attrT2, zr2, c2, embp, Wd, bd)

    NPAD = 10240
    zer = jnp.zeros((NPAD, H), jnp.float32)
    partials = _sc_scatter(col.reshape(NW, CPW, CK), msg, zer)

    return _tc_out(x, partials[0], partials[1], Wc[:H], Wc[H:], bc)


# speedup vs baseline: 7.0794x; 1.1102x over previous
import functools
from math import pi as PI

import jax
import jax.numpy as jnp
from jax import lax
from jax.experimental import pallas as pl
from jax.experimental.pallas import tpu as pltpu
from jax.experimental.pallas import tpu_sc as plsc

H = 128
NRBF = 16
NZ = 100
STOP = 5.0

NC = 2
NS = 16
NW = NC * NS

CK = 80
CPW = 125


def _sc_mesh():
    return plsc.VectorSubcoreMesh(core_axis_name="c", subcore_axis_name="s",
                                  num_cores=NC, num_subcores=NS)


def _zgather_body(z_hbm, ei_hbm, zrow_hbm, z_sh, idx_v, out_v, sem):
    cid_core = lax.axis_index("c")
    sid = lax.axis_index("s")
    wid = sid * NC + cid_core

    @pl.when(sid == 0)
    def _():
        pltpu.sync_copy(z_hbm, z_sh)

    pltpu.sync_copy(ei_hbm.at[0, wid], idx_v)
    plsc.subcore_barrier()

    BURST = 25
    for b0 in range(0, CPW, BURST):
        descs = [
            pltpu.async_copy(z_sh.at[idx_v.at[j]], out_v.at[j], sem)
            for j in range(b0, min(b0 + BURST, CPW))
        ]
        for d in descs:
            d.wait()
    pltpu.sync_copy(out_v, zrow_hbm.at[wid])


def _sc_zgather(z, ei4):
    mesh = _sc_mesh()
    return pl.kernel(
        _zgather_body,
        out_type=jax.ShapeDtypeStruct((NW, CPW, CK), jnp.int32),
        mesh=mesh,
        scratch_types=[
            pltpu.VMEM_SHARED((10000,), jnp.int32),
            pltpu.VMEM((CPW, CK), jnp.int32),
            pltpu.VMEM((CPW, CK), jnp.int32),
            pltpu.SemaphoreType.DMA,
        ],
    )(z, ei4)


def _msg_body(zr_ref, ew_ref, ei_ref, attrT_ref, embp_ref, wd_ref, bd_ref,
              out_ref):
    ng = zr_ref.shape[1]
    w = ew_ref[0]
    C = 0.5 * (jnp.cos(w * (PI / STOP)) + 1.0)
    C = C * (w < STOP).astype(jnp.float32)
    C = C * (ei_ref[0, 0] != ei_ref[1, 0]).astype(jnp.float32)
    ids = jax.lax.broadcasted_iota(jnp.int32, (H, H), 0)
    dn = (((0,), (0,)), ((), ()))
    zr = zr_ref[0]
    for g in range(ng):
        ohCT = (zr[g:g + 1, :] == ids).astype(jnp.float32) * C[g:g + 1, :]
        embrow = jax.lax.dot_general(ohCT, embp_ref[...], dn,
                                     preferred_element_type=jnp.float32)
        attrwT = attrT_ref[:, g * H:(g + 1) * H]
        attrw = jax.lax.dot_general(attrwT, wd_ref[...], dn,
                                    preferred_element_type=jnp.float32)
        out_ref[g * H:(g + 1) * H, :] = embrow * (attrw + bd_ref[...])


def _tc_msg(attrT, zr3, ew3, ei3, embp, Wd, bd):
    E = attrT.shape[1]
    BE = 2560
    GR = BE // H
    nb = E // BE
    return pl.pallas_call(
        _msg_body,
        grid=(nb,),
        in_specs=[
            pl.BlockSpec((1, GR, H), lambda i: (i, 0, 0)),
            pl.BlockSpec((1, GR, H), lambda i: (i, 0, 0)),
            pl.BlockSpec((2, 1, GR, H), lambda i: (0, i, 0, 0)),
            pl.BlockSpec((NRBF, BE), lambda i: (0, i)),
            pl.BlockSpec((H, H), lambda i: (0, 0)),
            pl.BlockSpec((NRBF, H), lambda i: (0, 0)),
            pl.BlockSpec((1, H), lambda i: (0, 0)),
        ],
        out_specs=pl.BlockSpec((BE, H), lambda i: (i, 0)),
        out_shape=jax.ShapeDtypeStruct((E, H), jnp.float32),
        compiler_params=pltpu.CompilerParams(
            dimension_semantics=("arbitrary",)),
    )(zr3, ew3, ei3, attrT, embp, Wd, bd.reshape(1, H))


def _scatter_body(ei_hbm, msg_hbm, zer_hbm, out_hbm, agg_sh, col_v, upd0,
                  upd1, upd2, lsem0, lsem1, lsem2, ssem0, ssem1, ssem2):
    NP = zer_hbm.shape[0]
    rows_per_tile = NP // NS
    cid_core = lax.axis_index("c")
    sid = lax.axis_index("s")
    wid = sid * NC + cid_core

    r0 = sid * rows_per_tile
    pltpu.sync_copy(zer_hbm.at[pl.ds(r0, rows_per_tile), :],
                    agg_sh.at[pl.ds(r0, rows_per_tile), :])
    pltpu.sync_copy(ei_hbm.at[1, wid], col_v)
    plsc.subcore_barrier()

    e0 = wid * (CPW * CK)
    bufs = (upd0, upd1, upd2)
    lsems = (lsem0, lsem1, lsem2)
    ssems = (ssem0, ssem1, ssem2)

    def load(j):
        return pltpu.async_copy(
            msg_hbm.at[pl.ds(e0 + j * CK, CK), :], bufs[j % 3], lsems[j % 3])

    def scat(j):
        return pltpu.async_copy(bufs[j % 3], agg_sh.at[col_v.at[j]],
                                ssems[j % 3], add=True)

    ld = [load(0), load(1), None]
    sc = [None, None, None]
    for j in range(CPW):
        b = j % 3
        ld[b].wait()
        sc[b] = scat(j)
        if j + 2 < CPW:
            b2 = (j + 2) % 3
            if sc[b2] is not None:
                sc[b2].wait()
            ld[b2] = load(j + 2)
    for d in sc:
        if d is not None:
            d.wait()

    plsc.subcore_barrier()
    pltpu.sync_copy(agg_sh.at[pl.ds(r0, rows_per_tile), :],
                    out_hbm.at[cid_core, pl.ds(r0, rows_per_tile), :])


def _sc_scatter(ei4, msg, zer):
    NP = zer.shape[0]
    mesh = _sc_mesh()
    return pl.kernel(
        _scatter_body,
        out_type=jax.ShapeDtypeStruct((NC, NP, H), jnp.float32),
        mesh=mesh,
        scratch_types=[
            pltpu.VMEM_SHARED((NP, H), jnp.float32),
            pltpu.VMEM((CPW, CK), jnp.int32),
            pltpu.VMEM((CK, H), jnp.float32),
            pltpu.VMEM((CK, H), jnp.float32),
            pltpu.VMEM((CK, H), jnp.float32),
            pltpu.SemaphoreType.DMA,
            pltpu.SemaphoreType.DMA,
            pltpu.SemaphoreType.DMA,
            pltpu.SemaphoreType.DMA,
            pltpu.SemaphoreType.DMA,
            pltpu.SemaphoreType.DMA,
        ],
    )(ei4, msg, zer)


def _out_body(x_ref, p0_ref, p1_ref, wct_ref, wcb_ref, bc_ref, out_ref):
    agg = p0_ref[...] + p1_ref[...]
    out_ref[...] = (
        jnp.dot(x_ref[...], wct_ref[...], preferred_element_type=jnp.float32)
        + jnp.dot(agg, wcb_ref[...], preferred_element_type=jnp.float32)
        + bc_ref[...])


def _tc_out(x, p0, p1, WcT, WcB, bc):
    N = x.shape[0]
    BN = 2000
    nb = N // BN
    col1 = lambda i: (i, 0)
    return pl.pallas_call(
        _out_body,
        grid=(nb,),
        in_specs=[
            pl.BlockSpec((BN, H), col1),
            pl.BlockSpec((BN, H), col1),
            pl.BlockSpec((BN, H), col1),
            pl.BlockSpec((H, H), lambda i: (0, 0)),
            pl.BlockSpec((H, H), lambda i: (0, 0)),
            pl.BlockSpec((1, H), lambda i: (0, 0)),
        ],
        out_specs=pl.BlockSpec((BN, H), col1),
        out_shape=jax.ShapeDtypeStruct((N, H), jnp.float32),
        compiler_params=pltpu.CompilerParams(
            dimension_semantics=("arbitrary",)),
    )(x, p0, p1, WcT, WcB, bc.reshape(1, H))


def kernel(z, x, edge_index, edge_weight, edge_attr, emb, Wd, bd, Wc, bc):
    N = x.shape[0]
    E = edge_index.shape[1]
    nr = E // H

    ei4 = edge_index.reshape(2, NW, CPW, CK)
    ei3 = edge_index.reshape(2, nr // 20, 20, H)
    ew3 = edge_weight.reshape(nr // 20, 20, H)

    zrow = _sc_zgather(z.astype(jnp.int32), ei4)
    zr3 = zrow.reshape(nr // 20, 20, H)

    embp = jnp.zeros((H, H), jnp.float32).at[:NZ, :].set(emb)
    msg = _tc_msg(edge_attr.T, zr3, ew3, ei3, embp, Wd, bd)

    NPAD = 10240
    zer = jnp.zeros((NPAD, H), jnp.float32)
    partials = _sc_scatter(ei4, msg, zer)

    return _tc_out(x, partials[0], partials[1], Wc[:H], Wc[H:], bc)


# speedup vs baseline: 7.3353x; 1.0361x over previous
import functools
from math import pi as PI

import jax
import jax.numpy as jnp
from jax import lax
from jax.experimental import pallas as pl
from jax.experimental.pallas import tpu as pltpu
from jax.experimental.pallas import tpu_sc as plsc

H = 128
NRBF = 16
NZ = 100
STOP = 5.0

NC = 2
NS = 16
NW = NC * NS

CKA = 80
CPWA = 125

NH = 2
CK = 40
CPW = 125

BE = 6400
GR = BE // H


def _sc_mesh():
    return plsc.VectorSubcoreMesh(core_axis_name="c", subcore_axis_name="s",
                                  num_cores=NC, num_subcores=NS)


def _zgather_body(z_hbm, ei_hbm, zrow_hbm, z_sh, idx_v, out_v, sem):
    cid_core = lax.axis_index("c")
    sid = lax.axis_index("s")
    wid = sid * NC + cid_core

    @pl.when(sid == 0)
    def _():
        pltpu.sync_copy(z_hbm, z_sh)

    pltpu.sync_copy(ei_hbm.at[0, wid], idx_v)
    plsc.subcore_barrier()

    BURST = 25
    for b0 in range(0, CPWA, BURST):
        descs = [
            pltpu.async_copy(z_sh.at[idx_v.at[j]], out_v.at[j], sem)
            for j in range(b0, min(b0 + BURST, CPWA))
        ]
        for d in descs:
            d.wait()
    pltpu.sync_copy(out_v, zrow_hbm.at[wid])


def _sc_zgather(z, ei4):
    mesh = _sc_mesh()
    return pl.kernel(
        _zgather_body,
        out_type=jax.ShapeDtypeStruct((NW, CPWA, CKA), jnp.int32),
        mesh=mesh,
        scratch_types=[
            pltpu.VMEM_SHARED((10000,), jnp.int32),
            pltpu.VMEM((CPWA, CKA), jnp.int32),
            pltpu.VMEM((CPWA, CKA), jnp.int32),
            pltpu.SemaphoreType.DMA,
        ],
    )(z, ei4)


def _msg_body(zr_ref, ew_ref, ei_ref, attrT_ref, embp_ref, wd_ref, bd_ref,
              out_ref):
    w = ew_ref[0]
    C = 0.5 * (jnp.cos(w * (PI / STOP)) + 1.0)
    C = C * (w < STOP).astype(jnp.float32)
    C = C * (ei_ref[0, 0] != ei_ref[1, 0]).astype(jnp.float32)
    ids = jax.lax.broadcasted_iota(jnp.int32, (H, H), 0)
    dn = (((0,), (0,)), ((), ()))
    zr = zr_ref[0]
    for g in range(GR):
        ohCT = (zr[g:g + 1, :] == ids).astype(jnp.float32) * C[g:g + 1, :]
        embrow = jax.lax.dot_general(ohCT, embp_ref[...], dn,
                                     preferred_element_type=jnp.float32)
        attrwT = attrT_ref[:, g * H:(g + 1) * H]
        attrw = jax.lax.dot_general(attrwT, wd_ref[...], dn,
                                    preferred_element_type=jnp.float32)
        out_ref[g * H:(g + 1) * H, :] = embrow * (attrw + bd_ref[...])


def _tc_msg(attrT, zr3, ew3, ei3, embp, Wd, bd, half):
    E = attrT.shape[1]
    Eh = E // NH
    nb = Eh // BE
    off = half * nb
    return pl.pallas_call(
        _msg_body,
        grid=(nb,),
        in_specs=[
            pl.BlockSpec((1, GR, H), lambda i: (i + off, 0, 0)),
            pl.BlockSpec((1, GR, H), lambda i: (i + off, 0, 0)),
            pl.BlockSpec((2, 1, GR, H), lambda i: (0, i + off, 0, 0)),
            pl.BlockSpec((NRBF, BE), lambda i: (0, i + off)),
            pl.BlockSpec((H, H), lambda i: (0, 0)),
            pl.BlockSpec((NRBF, H), lambda i: (0, 0)),
            pl.BlockSpec((1, H), lambda i: (0, 0)),
        ],
        out_specs=pl.BlockSpec((BE, H), lambda i: (i, 0)),
        out_shape=jax.ShapeDtypeStruct((Eh, H), jnp.float32),
        compiler_params=pltpu.CompilerParams(
            dimension_semantics=("arbitrary",)),
    )(zr3, ew3, ei3, attrT, embp, Wd, bd.reshape(1, H))


def _make_scatter_body(half):
    def _scatter_body(ei_hbm, msg_hbm, init_hbm, out_hbm, agg_sh, col_v,
                      upd0, upd1, upd2, lsem0, lsem1, lsem2, ssem0, ssem1,
                      ssem2):
        NP = init_hbm.shape[1]
        rows_per_tile = NP // NS
        cid_core = lax.axis_index("c")
        sid = lax.axis_index("s")
        wid = sid * NC + cid_core

        r0 = sid * rows_per_tile
        pltpu.sync_copy(init_hbm.at[cid_core, pl.ds(r0, rows_per_tile), :],
                        agg_sh.at[pl.ds(r0, rows_per_tile), :])
        pltpu.sync_copy(ei_hbm.at[1, half, wid], col_v)
        plsc.subcore_barrier()

        e0 = wid * (CPW * CK)
        bufs = (upd0, upd1, upd2)
        lsems = (lsem0, lsem1, lsem2)
        ssems = (ssem0, ssem1, ssem2)

        def load(j):
            return pltpu.async_copy(
                msg_hbm.at[pl.ds(e0 + j * CK, CK), :], bufs[j % 3],
                lsems[j % 3])

        def scat(j):
            return pltpu.async_copy(bufs[j % 3], agg_sh.at[col_v.at[j]],
                                    ssems[j % 3], add=True)

        ld = [load(0), load(1), None]
        sc = [None, None, None]
        for j in range(CPW):
            b = j % 3
            ld[b].wait()
            sc[b] = scat(j)
            if j + 2 < CPW:
                b2 = (j + 2) % 3
                if sc[b2] is not None:
                    sc[b2].wait()
                ld[b2] = load(j + 2)
        for d in sc:
            if d is not None:
                d.wait()

        plsc.subcore_barrier()
        pltpu.sync_copy(agg_sh.at[pl.ds(r0, rows_per_tile), :],
                        out_hbm.at[cid_core, pl.ds(r0, rows_per_tile), :])

    return _scatter_body


def _sc_scatter(ei5, msg_h, init, half):
    NP = init.shape[1]
    mesh = _sc_mesh()
    return pl.kernel(
        _make_scatter_body(half),
        out_type=jax.ShapeDtypeStruct((NC, NP, H), jnp.float32),
        mesh=mesh,
        scratch_types=[
            pltpu.VMEM_SHARED((NP, H), jnp.float32),
            pltpu.VMEM((CPW, CK), jnp.int32),
            pltpu.VMEM((CK, H), jnp.float32),
            pltpu.VMEM((CK, H), jnp.float32),
            pltpu.VMEM((CK, H), jnp.float32),
            pltpu.SemaphoreType.DMA,
            pltpu.SemaphoreType.DMA,
            pltpu.SemaphoreType.DMA,
            pltpu.SemaphoreType.DMA,
            pltpu.SemaphoreType.DMA,
            pltpu.SemaphoreType.DMA,
        ],
    )(ei5, msg_h, init)


def _out_body(x_ref, p0_ref, p1_ref, wct_ref, wcb_ref, bc_ref, out_ref):
    agg = p0_ref[...] + p1_ref[...]
    out_ref[...] = (
        jnp.dot(x_ref[...], wct_ref[...], preferred_element_type=jnp.float32)
        + jnp.dot(agg, wcb_ref[...], preferred_element_type=jnp.float32)
        + bc_ref[...])


def _tc_out(x, p0, p1, WcT, WcB, bc):
    N = x.shape[0]
    BN = 2000
    nb = N // BN
    col1 = lambda i: (i, 0)
    return pl.pallas_call(
        _out_body,
        grid=(nb,),
        in_specs=[
            pl.BlockSpec((BN, H), col1),
            pl.BlockSpec((BN, H), col1),
            pl.BlockSpec((BN, H), col1),
            pl.BlockSpec((H, H), lambda i: (0, 0)),
            pl.BlockSpec((H, H), lambda i: (0, 0)),
            pl.BlockSpec((1, H), lambda i: (0, 0)),
        ],
        out_specs=pl.BlockSpec((BN, H), col1),
        out_shape=jax.ShapeDtypeStruct((N, H), jnp.float32),
        compiler_params=pltpu.CompilerParams(
            dimension_semantics=("arbitrary",)),
    )(x, p0, p1, WcT, WcB, bc.reshape(1, H))


def kernel(z, x, edge_index, edge_weight, edge_attr, emb, Wd, bd, Wc, bc):
    N = x.shape[0]
    E = edge_index.shape[1]
    nr = E // H
    nb3 = E // BE

    ei4 = edge_index.reshape(2, NW, CPWA, CKA)
    ei5 = edge_index.reshape(2, NH, NW, CPW, CK)
    ei3 = edge_index.reshape(2, nb3, GR, H)
    ew3 = edge_weight.reshape(nb3, GR, H)

    zrow = _sc_zgather(z.astype(jnp.int32), ei4)
    zr3 = zrow.reshape(nb3, GR, H)

    embp = jnp.zeros((H, H), jnp.float32).at[:NZ, :].set(emb)
    attrT = edge_attr.T

    NPAD = 10240
    zer = jnp.zeros((NC, NPAD, H), jnp.float32)

    msg0 = _tc_msg(attrT, zr3, ew3, ei3, embp, Wd, bd, 0)
    part0 = _sc_scatter(ei5, msg0, zer, 0)
    msg1 = _tc_msg(attrT, zr3, ew3, ei3, embp, Wd, bd, 1)
    part1 = _sc_scatter(ei5, msg1, part0, 1)

    return _tc_out(x, part1[0], part1[1], Wc[:H], Wc[H:], bc)


# speedup vs baseline: 7.8263x; 1.0669x over previous
import functools
from math import pi as PI

import jax
import jax.numpy as jnp
from jax import lax
from jax.experimental import pallas as pl
from jax.experimental.pallas import tpu as pltpu
from jax.experimental.pallas import tpu_sc as plsc

H = 128
NRBF = 16
NZ = 100
STOP = 5.0

NC = 2
NS = 16
NW = NC * NS

E2 = 327680
CKA = 80
CPWA = 128

NH = 2
CK = 80
CPW = 64

BE = 8192
GR = BE // H


def _sc_mesh():
    return plsc.VectorSubcoreMesh(core_axis_name="c", subcore_axis_name="s",
                                  num_cores=NC, num_subcores=NS)


def _zgather_body(z_hbm, ei_hbm, zrow_hbm, z_sh, idx_v, out_v, sem):
    cid_core = lax.axis_index("c")
    sid = lax.axis_index("s")
    wid = sid * NC + cid_core

    @pl.when(sid == 0)
    def _():
        pltpu.sync_copy(z_hbm, z_sh)

    pltpu.sync_copy(ei_hbm.at[0, wid], idx_v)
    plsc.subcore_barrier()

    BURST = 25
    for b0 in range(0, CPWA, BURST):
        descs = [
            pltpu.async_copy(z_sh.at[idx_v.at[j]], out_v.at[j], sem)
            for j in range(b0, min(b0 + BURST, CPWA))
        ]
        for d in descs:
            d.wait()
    pltpu.sync_copy(out_v, zrow_hbm.at[wid])


def _sc_zgather(z, ei4):
    mesh = _sc_mesh()
    return pl.kernel(
        _zgather_body,
        out_type=jax.ShapeDtypeStruct((NW, CPWA, CKA), jnp.int32),
        mesh=mesh,
        scratch_types=[
            pltpu.VMEM_SHARED((10000,), jnp.int32),
            pltpu.VMEM((CPWA, CKA), jnp.int32),
            pltpu.VMEM((CPWA, CKA), jnp.int32),
            pltpu.SemaphoreType.DMA,
        ],
    )(z, ei4)


def _msg_body(zr_ref, ew_ref, ei_ref, attrT_ref, embp_ref, wd_ref, bd_ref,
              out_ref):
    w = ew_ref[0]
    C = 0.5 * (jnp.cos(w * (PI / STOP)) + 1.0)
    C = C * (w < STOP).astype(jnp.float32)
    C = C * (ei_ref[0, 0] != ei_ref[1, 0]).astype(jnp.float32)
    ids = jax.lax.broadcasted_iota(jnp.int32, (H, H), 0)
    dn = (((0,), (0,)), ((), ()))
    zr = zr_ref[0]
    for g in range(GR):
        ohCT = (zr[g:g + 1, :] == ids).astype(jnp.float32) * C[g:g + 1, :]
        embrow = jax.lax.dot_general(ohCT, embp_ref[...], dn,
                                     preferred_element_type=jnp.float32)
        attrwT = attrT_ref[:, g * H:(g + 1) * H]
        attrw = jax.lax.dot_general(attrwT, wd_ref[...], dn,
                                    preferred_element_type=jnp.float32)
        out_ref[g * H:(g + 1) * H, :] = embrow * (attrw + bd_ref[...])


def _tc_msg(attrT, zr3, ew3, ei3, embp, Wd, bd, half):
    Eh = E2 // NH
    nb = Eh // BE
    off = half * nb
    return pl.pallas_call(
        _msg_body,
        grid=(nb,),
        in_specs=[
            pl.BlockSpec((1, GR, H), lambda i: (i + off, 0, 0)),
            pl.BlockSpec((1, GR, H), lambda i: (i + off, 0, 0)),
            pl.BlockSpec((2, 1, GR, H), lambda i: (0, i + off, 0, 0)),
            pl.BlockSpec((NRBF, BE), lambda i: (0, i + off)),
            pl.BlockSpec((H, H), lambda i: (0, 0)),
            pl.BlockSpec((NRBF, H), lambda i: (0, 0)),
            pl.BlockSpec((1, H), lambda i: (0, 0)),
        ],
        out_specs=pl.BlockSpec((BE, H), lambda i: (i, 0)),
        out_shape=jax.ShapeDtypeStruct((Eh, H), jnp.float32),
        compiler_params=pltpu.CompilerParams(
            dimension_semantics=("arbitrary",)),
    )(zr3, ew3, ei3, attrT, embp, Wd, bd.reshape(1, H))


def _make_scatter_body(half):
    def _scatter_body(ei_hbm, msg_hbm, init_hbm, out_hbm, agg_sh, col_v,
                      upd0, upd1, upd2, lsem0, lsem1, lsem2, ssem0, ssem1,
                      ssem2):
        NP = init_hbm.shape[1]
        rows_per_tile = NP // NS
        cid_core = lax.axis_index("c")
        sid = lax.axis_index("s")
        wid = sid * NC + cid_core

        r0 = sid * rows_per_tile
        pltpu.sync_copy(init_hbm.at[cid_core, pl.ds(r0, rows_per_tile), :],
                        agg_sh.at[pl.ds(r0, rows_per_tile), :])
        pltpu.sync_copy(ei_hbm.at[1, half, wid], col_v)
        plsc.subcore_barrier()

        e0 = wid * (CPW * CK)
        bufs = (upd0, upd1, upd2)
        lsems = (lsem0, lsem1, lsem2)
        ssems = (ssem0, ssem1, ssem2)

        def load(j):
            return pltpu.async_copy(
                msg_hbm.at[pl.ds(e0 + j * CK, CK), :], bufs[j % 3],
                lsems[j % 3])

        def scat(j):
            return pltpu.async_copy(bufs[j % 3], agg_sh.at[col_v.at[j]],
                                    ssems[j % 3], add=True)

        ld = [load(0), load(1), None]
        sc = [None, None, None]
        for j in range(CPW):
            b = j % 3
            ld[b].wait()
            sc[b] = scat(j)
            if j + 2 < CPW:
                b2 = (j + 2) % 3
                if sc[b2] is not None:
                    sc[b2].wait()
                ld[b2] = load(j + 2)
        for d in sc:
            if d is not None:
                d.wait()

        plsc.subcore_barrier()
        pltpu.sync_copy(agg_sh.at[pl.ds(r0, rows_per_tile), :],
                        out_hbm.at[cid_core, pl.ds(r0, rows_per_tile), :])

    return _scatter_body


def _sc_scatter(ei5, msg_h, init, half):
    NP = init.shape[1]
    mesh = _sc_mesh()
    return pl.kernel(
        _make_scatter_body(half),
        out_type=jax.ShapeDtypeStruct((NC, NP, H), jnp.float32),
        mesh=mesh,
        scratch_types=[
            pltpu.VMEM_SHARED((NP, H), jnp.float32),
            pltpu.VMEM((CPW, CK), jnp.int32),
            pltpu.VMEM((CK, H), jnp.float32),
            pltpu.VMEM((CK, H), jnp.float32),
            pltpu.VMEM((CK, H), jnp.float32),
            pltpu.SemaphoreType.DMA,
            pltpu.SemaphoreType.DMA,
            pltpu.SemaphoreType.DMA,
            pltpu.SemaphoreType.DMA,
            pltpu.SemaphoreType.DMA,
            pltpu.SemaphoreType.DMA,
        ],
    )(ei5, msg_h, init)


def _out_body(x_ref, p0_ref, p1_ref, wct_ref, wcb_ref, bc_ref, out_ref):
    agg = p0_ref[...] + p1_ref[...]
    out_ref[...] = (
        jnp.dot(x_ref[...], wct_ref[...], preferred_element_type=jnp.float32)
        + jnp.dot(agg, wcb_ref[...], preferred_element_type=jnp.float32)
        + bc_ref[...])


def _tc_out(x, p0, p1, WcT, WcB, bc):
    N = x.shape[0]
    BN = 2000
    nb = N // BN
    col1 = lambda i: (i, 0)
    return pl.pallas_call(
        _out_body,
        grid=(nb,),
        in_specs=[
            pl.BlockSpec((BN, H), col1),
            pl.BlockSpec((BN, H), col1),
            pl.BlockSpec((BN, H), col1),
            pl.BlockSpec((H, H), lambda i: (0, 0)),
            pl.BlockSpec((H, H), lambda i: (0, 0)),
            pl.BlockSpec((1, H), lambda i: (0, 0)),
        ],
        out_specs=pl.BlockSpec((BN, H), col1),
        out_shape=jax.ShapeDtypeStruct((N, H), jnp.float32),
        compiler_params=pltpu.CompilerParams(
            dimension_semantics=("arbitrary",)),
    )(x, p0, p1, WcT, WcB, bc.reshape(1, H))


def kernel(z, x, edge_index, edge_weight, edge_attr, emb, Wd, bd, Wc, bc):
    N = x.shape[0]
    E = edge_index.shape[1]
    padE = E2 - E
    nb3 = E2 // BE

    eip = jnp.pad(edge_index, ((0, 0), (0, padE)))
    ewp = jnp.pad(edge_weight, (0, padE))
    ei4 = eip.reshape(2, NW, CPWA, CKA)
    ei5 = eip.reshape(2, NH, NW, CPW, CK)
    ei3 = eip.reshape(2, nb3, GR, H)
    ew3 = ewp.reshape(nb3, GR, H)

    zrow = _sc_zgather(z.astype(jnp.int32), ei4)
    zr3 = zrow.reshape(nb3, GR, H)

    embp = jnp.zeros((H, H), jnp.float32).at[:NZ, :].set(emb)
    attrT = jnp.pad(edge_attr.T, ((0, 0), (0, padE)))

    NPAD = 10240
    zer = jnp.zeros((NC, NPAD, H), jnp.float32)

    msg0 = _tc_msg(attrT, zr3, ew3, ei3, embp, Wd, bd, 0)
    part0 = _sc_scatter(ei5, msg0, zer, 0)
    msg1 = _tc_msg(attrT, zr3, ew3, ei3, embp, Wd, bd, 1)
    part1 = _sc_scatter(ei5, msg1, part0, 1)

    return _tc_out(x, part1[0], part1[1], Wc[:H], Wc[H:], bc)


# speedup vs baseline: 8.2259x; 1.0511x over previous
import functools
from math import pi as PI

import jax
import jax.numpy as jnp
from jax import lax
from jax.experimental import pallas as pl
from jax.experimental.pallas import tpu as pltpu
from jax.experimental.pallas import tpu_sc as plsc

H = 128
NRBF = 16
NZ = 100
STOP = 5.0

NC = 2
NS = 16
NW = NC * NS

E2 = 327680
CKA = 128
CPWA = 80

NH = 2
CK = 80
CPW = 64

BE = 8192
GR = BE // H


def _sc_mesh():
    return plsc.VectorSubcoreMesh(core_axis_name="c", subcore_axis_name="s",
                                  num_cores=NC, num_subcores=NS)


def _zgather_body(z_hbm, ei_hbm, zrow_hbm, z_sh, idx_v, out_v, sem):
    cid_core = lax.axis_index("c")
    sid = lax.axis_index("s")
    wid = sid * NC + cid_core

    @pl.when(sid == 0)
    def _():
        pltpu.sync_copy(z_hbm, z_sh)

    pltpu.sync_copy(ei_hbm.at[0, wid], idx_v)
    plsc.subcore_barrier()

    BURST = 25
    for b0 in range(0, CPWA, BURST):
        descs = [
            pltpu.async_copy(z_sh.at[idx_v.at[j]], out_v.at[j], sem)
            for j in range(b0, min(b0 + BURST, CPWA))
        ]
        for d in descs:
            d.wait()
    pltpu.sync_copy(out_v, zrow_hbm.at[pl.ds(wid * CPWA, CPWA), :])


def _sc_zgather(z, ei4):
    mesh = _sc_mesh()
    return pl.kernel(
        _zgather_body,
        out_type=jax.ShapeDtypeStruct((NW * CPWA, CKA), jnp.int32),
        mesh=mesh,
        scratch_types=[
            pltpu.VMEM_SHARED((10000,), jnp.int32),
            pltpu.VMEM((CPWA, CKA), jnp.int32),
            pltpu.VMEM((CPWA, CKA), jnp.int32),
            pltpu.SemaphoreType.DMA,
        ],
    )(z, ei4)


def _msg_body(zr_ref, ew_ref, ei_ref, attrT_ref, embp_ref, wd_ref, bd_ref,
              out_ref):
    w = ew_ref[0]
    C = 0.5 * (jnp.cos(w * (PI / STOP)) + 1.0)
    C = C * (w < STOP).astype(jnp.float32)
    C = C * (ei_ref[0, 0] != ei_ref[1, 0]).astype(jnp.float32)
    ids = jax.lax.broadcasted_iota(jnp.int32, (H, H), 0)
    dn = (((0,), (0,)), ((), ()))
    zr = zr_ref[0]
    for g in range(GR):
        ohCT = (zr[g:g + 1, :] == ids).astype(jnp.float32) * C[g:g + 1, :]
        embrow = jax.lax.dot_general(ohCT, embp_ref[...], dn,
                                     preferred_element_type=jnp.float32)
        attrwT = attrT_ref[:, g * H:(g + 1) * H]
        attrw = jax.lax.dot_general(attrwT, wd_ref[...], dn,
                                    preferred_element_type=jnp.float32)
        out_ref[g * H:(g + 1) * H, :] = embrow * (attrw + bd_ref[...])


def _tc_msg(attrT, zr3, ew3, ei3, embp, Wd, bd, half):
    Eh = E2 // NH
    nb = Eh // BE
    off = half * nb
    return pl.pallas_call(
        _msg_body,
        grid=(nb,),
        in_specs=[
            pl.BlockSpec((1, GR, H), lambda i: (i + off, 0, 0)),
            pl.BlockSpec((1, GR, H), lambda i: (i + off, 0, 0)),
            pl.BlockSpec((2, 1, GR, H), lambda i: (0, i + off, 0, 0)),
            pl.BlockSpec((NRBF, BE), lambda i: (0, i + off)),
            pl.BlockSpec((H, H), lambda i: (0, 0)),
            pl.BlockSpec((NRBF, H), lambda i: (0, 0)),
            pl.BlockSpec((1, H), lambda i: (0, 0)),
        ],
        out_specs=pl.BlockSpec((BE, H), lambda i: (i, 0)),
        out_shape=jax.ShapeDtypeStruct((Eh, H), jnp.float32),
        compiler_params=pltpu.CompilerParams(
            dimension_semantics=("arbitrary",)),
    )(zr3, ew3, ei3, attrT, embp, Wd, bd.reshape(1, H))


def _make_scatter_body(half):
    def _scatter_body(ei_hbm, msg_hbm, init_hbm, out_hbm, agg_sh, col_v,
                      upd0, upd1, upd2, upd3, lsem0, lsem1, lsem2, lsem3,
                      ssem0, ssem1, ssem2, ssem3):
        NP = init_hbm.shape[1]
        rows_per_tile = NP // NS
        cid_core = lax.axis_index("c")
        sid = lax.axis_index("s")
        wid = sid * NC + cid_core

        r0 = sid * rows_per_tile
        pltpu.sync_copy(init_hbm.at[cid_core, pl.ds(r0, rows_per_tile), :],
                        agg_sh.at[pl.ds(r0, rows_per_tile), :])
        pltpu.sync_copy(ei_hbm.at[1, half, wid], col_v)
        plsc.subcore_barrier()

        e0 = wid * (CPW * CK)
        NB = 4
        bufs = (upd0, upd1, upd2, upd3)
        lsems = (lsem0, lsem1, lsem2, lsem3)
        ssems = (ssem0, ssem1, ssem2, ssem3)

        def load(j):
            return pltpu.async_copy(
                msg_hbm.at[pl.ds(e0 + j * CK, CK), :], bufs[j % NB],
                lsems[j % NB])

        def scat(j):
            return pltpu.async_copy(bufs[j % NB], agg_sh.at[col_v.at[j]],
                                    ssems[j % NB], add=True)

        ld = [load(0), load(1), load(2), None]
        sc = [None, None, None, None]
        for j in range(CPW):
            b = j % NB
            ld[b].wait()
            sc[b] = scat(j)
            if j + 3 < CPW:
                b2 = (j + 3) % NB
                if sc[b2] is not None:
                    sc[b2].wait()
                ld[b2] = load(j + 3)
        for d in sc:
            if d is not None:
                d.wait()

        plsc.subcore_barrier()
        pltpu.sync_copy(agg_sh.at[pl.ds(r0, rows_per_tile), :],
                        out_hbm.at[cid_core, pl.ds(r0, rows_per_tile), :])

    return _scatter_body


def _sc_scatter(ei5, msg_h, init, half):
    NP = init.shape[1]
    mesh = _sc_mesh()
    return pl.kernel(
        _make_scatter_body(half),
        out_type=jax.ShapeDtypeStruct((NC, NP, H), jnp.float32),
        mesh=mesh,
        scratch_types=[
            pltpu.VMEM_SHARED((NP, H), jnp.float32),
            pltpu.VMEM((CPW, CK), jnp.int32),
            pltpu.VMEM((CK, H), jnp.float32),
            pltpu.VMEM((CK, H), jnp.float32),
            pltpu.VMEM((CK, H), jnp.float32),
            pltpu.VMEM((CK, H), jnp.float32),
            pltpu.SemaphoreType.DMA,
            pltpu.SemaphoreType.DMA,
            pltpu.SemaphoreType.DMA,
            pltpu.SemaphoreType.DMA,
            pltpu.SemaphoreType.DMA,
            pltpu.SemaphoreType.DMA,
            pltpu.SemaphoreType.DMA,
            pltpu.SemaphoreType.DMA,
        ],
    )(ei5, msg_h, init)


def _out_body(x_ref, p0_ref, p1_ref, wct_ref, wcb_ref, bc_ref, out_ref):
    agg = p0_ref[0] + p1_ref[0]
    out_ref[...] = (
        jnp.dot(x_ref[...], wct_ref[...], preferred_element_type=jnp.float32)
        + jnp.dot(agg, wcb_ref[...], preferred_element_type=jnp.float32)
        + bc_ref[...])


def _tc_out(x, parts, WcT, WcB, bc):
    N = x.shape[0]
    BN = 2000
    nb = N // BN
    col1 = lambda i: (i, 0)
    return pl.pallas_call(
        _out_body,
        grid=(nb,),
        in_specs=[
            pl.BlockSpec((BN, H), col1),
            pl.BlockSpec((1, BN, H), lambda i: (0, i, 0)),
            pl.BlockSpec((1, BN, H), lambda i: (1, i, 0)),
            pl.BlockSpec((H, H), lambda i: (0, 0)),
            pl.BlockSpec((H, H), lambda i: (0, 0)),
            pl.BlockSpec((1, H), lambda i: (0, 0)),
        ],
        out_specs=pl.BlockSpec((BN, H), col1),
        out_shape=jax.ShapeDtypeStruct((N, H), jnp.float32),
        compiler_params=pltpu.CompilerParams(
            dimension_semantics=("arbitrary",)),
    )(x, parts, parts, WcT, WcB, bc.reshape(1, H))


def kernel(z, x, edge_index, edge_weight, edge_attr, emb, Wd, bd, Wc, bc):
    N = x.shape[0]
    E = edge_index.shape[1]
    padE = E2 - E
    nb3 = E2 // BE

    eip = jnp.pad(edge_index, ((0, 0), (0, padE)))
    ewp = jnp.pad(edge_weight, (0, padE))
    ei4 = eip.reshape(2, NW, CPWA, CKA)
    ei5 = eip.reshape(2, NH, NW, CPW, CK)
    ei3 = eip.reshape(2, nb3, GR, H)
    ew3 = ewp.reshape(nb3, GR, H)

    zrow = _sc_zgather(z.astype(jnp.int32), ei4)
    zr3 = zrow.reshape(nb3, GR, H)

    embp = jnp.zeros((H, H), jnp.float32).at[:NZ, :].set(emb)
    attrT = jnp.pad(edge_attr.T, ((0, 0), (0, padE)))

    NPAD = 10240
    zer = jnp.zeros((NC, NPAD, H), jnp.float32)

    msg0 = _tc_msg(attrT, zr3, ew3, ei3, embp, Wd, bd, 0)
    part0 = _sc_scatter(ei5, msg0, zer, 0)
    msg1 = _tc_msg(attrT, zr3, ew3, ei3, embp, Wd, bd, 1)
    part1 = _sc_scatter(ei5, msg1, part0, 1)

    return _tc_out(x, part1, Wc[:H], Wc[H:], bc)
